# baseline (speedup 1.0000x reference)
import functools

import jax
import jax.numpy as jnp
from jax.experimental import pallas as pl
from jax.experimental.pallas import tpu as pltpu

N = 4096
D = 256
BM = 512
BK = 1024


def _xw_body(x_ref, w_ref, out_ref):
    out_ref[...] = jnp.dot(
        x_ref[...].astype(jnp.bfloat16),
        w_ref[...].astype(jnp.bfloat16),
        preferred_element_type=jnp.float32,
    )


def _first_matmul(x, w):
    n, d_in = x.shape
    d_out = w.shape[1]
    return pl.pallas_call(
        _xw_body,
        grid=(n // BM,),
        in_specs=[
            pl.BlockSpec((BM, d_in), lambda i: (i, 0)),
            pl.BlockSpec((d_in, d_out), lambda i: (0, 0)),
        ],
        out_specs=pl.BlockSpec((BM, d_out), lambda i: (i, 0)),
        out_shape=jax.ShapeDtypeStruct((n, d_out), jnp.float32),
    )(x, w)


def _layer_body(adj_ref, xw_ref, b_ref, wn_ref, out_ref, acc_ref, *, fuse_next):
    k = pl.program_id(1)
    nk = pl.num_programs(1)
    prod = jnp.dot(
        adj_ref[...].astype(jnp.bfloat16),
        xw_ref[...].astype(jnp.bfloat16),
        preferred_element_type=jnp.float32,
    )

    @pl.when(k == 0)
    def _():
        acc_ref[...] = prod

    @pl.when(k > 0)
    def _():
        acc_ref[...] += prod

    @pl.when(k == nk - 1)
    def _():
        h = jnp.maximum(acc_ref[...] + b_ref[...], 0.0)
        if fuse_next:
            out_ref[...] = jnp.dot(
                h.astype(jnp.bfloat16),
                wn_ref[...].astype(jnp.bfloat16),
                preferred_element_type=jnp.float32,
            )
        else:
            out_ref[...] = h


def _layer(adj, xw, b, w_next):
    n = adj.shape[0]
    d = xw.shape[1]
    fuse_next = w_next is not None
    if not fuse_next:
        w_next = jnp.zeros((d, d), jnp.float32)
    body = functools.partial(_layer_body, fuse_next=fuse_next)
    return pl.pallas_call(
        body,
        grid=(n // BM, n // BK),
        in_specs=[
            pl.BlockSpec((BM, BK), lambda i, k: (i, k)),
            pl.BlockSpec((BK, d), lambda i, k: (k, 0)),
            pl.BlockSpec((1, d), lambda i, k: (0, 0)),
            pl.BlockSpec((d, d), lambda i, k: (0, 0)),
        ],
        out_specs=pl.BlockSpec((BM, d), lambda i, k: (i, 0)),
        out_shape=jax.ShapeDtypeStruct((n, d), jnp.float32),
        scratch_shapes=[pltpu.VMEM((BM, d), jnp.float32)],
        compiler_params=pltpu.CompilerParams(
            dimension_semantics=("parallel", "arbitrary"),
        ),
    )(adj, xw, b.reshape(1, d), w_next)


@jax.jit
def kernel(x, adj, W1, b1, W2, b2, W3, b3):
    xw0 = _first_matmul(x, W1)
    xw1 = _layer(adj, xw0, b1, W2)
    xw2 = _layer(adj, xw1, b2, W3)
    return _layer(adj, xw2, b3, None)


# speedup vs baseline: 1.9110x; 1.9110x over previous
import jax
import jax.numpy as jnp
from jax.experimental import pallas as pl
from jax.experimental.pallas import tpu as pltpu

N = 4096
D = 256
BM = 256
I = N // BM


def _body(x_ref, adj_ref, w1_ref, wn_ref, b_ref, out_ref,
          adjbf_ref, xwa_ref, xwb_ref):
    p = pl.program_id(0)
    i = pl.program_id(1)
    r = pl.ds(i * BM, BM)

    @pl.when(p == 0)
    def _():
        xwb_ref[r, :] = jnp.dot(
            x_ref[...].astype(jnp.bfloat16), w1_ref[...],
            preferred_element_type=jnp.float32,
        ).astype(jnp.bfloat16)

    @pl.when(p == 1)
    def _():
        ab = adj_ref[...].astype(jnp.bfloat16)
        adjbf_ref[r, :] = ab
        acc = jnp.dot(ab, xwb_ref[...], preferred_element_type=jnp.float32)
        h = jnp.maximum(acc + b_ref[0], 0.0).astype(jnp.bfloat16)
        xwa_ref[r, :] = jnp.dot(
            h, wn_ref[0], preferred_element_type=jnp.float32
        ).astype(jnp.bfloat16)

    @pl.when(p == 2)
    def _():
        a = adjbf_ref[r, :]
        acc = jnp.dot(a, xwa_ref[...], preferred_element_type=jnp.float32)
        h = jnp.maximum(acc + b_ref[0], 0.0).astype(jnp.bfloat16)
        xwb_ref[r, :] = jnp.dot(
            h, wn_ref[0], preferred_element_type=jnp.float32
        ).astype(jnp.bfloat16)

    @pl.when(p == 3)
    def _():
        a = adjbf_ref[r, :]
        acc = jnp.dot(a, xwb_ref[...], preferred_element_type=jnp.float32)
        out_ref[...] = jnp.maximum(acc + b_ref[0], 0.0)


@jax.jit
def kernel(x, adj, W1, b1, W2, b2, W3, b3):
    w1 = W1.astype(jnp.bfloat16)
    wn = jnp.stack([W2, W3]).astype(jnp.bfloat16)
    b = jnp.stack([b1, b2, b3]).reshape(3, 1, D)

    grid = (4, I)
    last = I - 1
    return pl.pallas_call(
        _body,
        grid=grid,
        in_specs=[
            pl.BlockSpec((BM, D), lambda p, i: (jnp.where(p == 0, i, last), 0)),
            pl.BlockSpec(
                (BM, N),
                lambda p, i: (jnp.where(p == 1, i, jnp.where(p < 1, 0, last)), 0),
            ),
            pl.BlockSpec((D, D), lambda p, i: (0, 0)),
            pl.BlockSpec((1, D, D), lambda p, i: (jnp.clip(p - 1, 0, 1), 0, 0)),
            pl.BlockSpec((1, 1, D), lambda p, i: (jnp.clip(p - 1, 0, 2), 0, 0)),
        ],
        out_specs=pl.BlockSpec((BM, D), lambda p, i: (jnp.where(p == 3, i, 0), 0)),
        out_shape=jax.ShapeDtypeStruct((N, D), jnp.float32),
        scratch_shapes=[
            pltpu.VMEM((N, N), jnp.bfloat16),
            pltpu.VMEM((N, D), jnp.bfloat16),
            pltpu.VMEM((N, D), jnp.bfloat16),
        ],
        compiler_params=pltpu.CompilerParams(
            dimension_semantics=("arbitrary", "arbitrary"),
        ),
    )(x, adj, w1, wn, b)


# speedup vs baseline: 2.4668x; 1.2909x over previous
import jax
import jax.numpy as jnp
from jax.experimental import pallas as pl
from jax.experimental.pallas import tpu as pltpu

N = 4096
D = 256
BM = 512
I = N // BM


def _body(x_ref, adj_ref, w1_ref, wn_ref, b_ref, out_ref,
          adjbf_ref, xwa_ref, xwb_ref):
    p = pl.program_id(0)
    i = pl.program_id(1)
    r = pl.ds(i * BM, BM)

    @pl.when((p == 0) & (i == 0))
    def _():
        xwb_ref[...] = jnp.dot(
            x_ref[...], w1_ref[...], preferred_element_type=jnp.float32
        ).astype(jnp.bfloat16)

    @pl.when(p == 0)
    def _():
        ab = adj_ref[...].astype(jnp.bfloat16)
        adjbf_ref[r, :] = ab
        acc = jnp.dot(ab, xwb_ref[...], preferred_element_type=jnp.float32)
        h = jnp.maximum(acc + b_ref[0], 0.0).astype(jnp.bfloat16)
        xwa_ref[r, :] = jnp.dot(
            h, wn_ref[0], preferred_element_type=jnp.float32
        ).astype(jnp.bfloat16)

    @pl.when(p == 1)
    def _():
        a = adjbf_ref[r, :]
        acc = jnp.dot(a, xwa_ref[...], preferred_element_type=jnp.float32)
        h = jnp.maximum(acc + b_ref[0], 0.0).astype(jnp.bfloat16)
        xwb_ref[r, :] = jnp.dot(
            h, wn_ref[0], preferred_element_type=jnp.float32
        ).astype(jnp.bfloat16)

    @pl.when(p == 2)
    def _():
        a = adjbf_ref[r, :]
        acc = jnp.dot(a, xwb_ref[...], preferred_element_type=jnp.float32)
        out_ref[...] = jnp.maximum(acc + b_ref[0], 0.0)


@jax.jit
def kernel(x, adj, W1, b1, W2, b2, W3, b3):
    xbf = x.astype(jnp.bfloat16)
    w1 = W1.astype(jnp.bfloat16)
    wn = jnp.stack([W2, W3]).astype(jnp.bfloat16)
    b = jnp.stack([b1, b2, b3]).reshape(3, 1, D)

    last = I - 1
    return pl.pallas_call(
        _body,
        grid=(3, I),
        in_specs=[
            pl.BlockSpec((N, D), lambda p, i: (0, 0)),
            pl.BlockSpec((BM, N), lambda p, i: (jnp.where(p == 0, i, last), 0)),
            pl.BlockSpec((D, D), lambda p, i: (0, 0)),
            pl.BlockSpec((1, D, D), lambda p, i: (jnp.clip(p, 0, 1), 0, 0)),
            pl.BlockSpec((1, 1, D), lambda p, i: (p, 0, 0)),
        ],
        out_specs=pl.BlockSpec((BM, D), lambda p, i: (jnp.where(p == 2, i, 0), 0)),
        out_shape=jax.ShapeDtypeStruct((N, D), jnp.float32),
        scratch_shapes=[
            pltpu.VMEM((N, N), jnp.bfloat16),
            pltpu.VMEM((N, D), jnp.bfloat16),
            pltpu.VMEM((N, D), jnp.bfloat16),
        ],
        compiler_params=pltpu.CompilerParams(
            dimension_semantics=("arbitrary", "arbitrary"),
        ),
    )(xbf, adj, w1, wn, b)
